# baseline (speedup 1.0000x reference)
import jax
import jax.numpy as jnp
from jax import lax
from jax.experimental import pallas as pl
from jax.experimental.pallas import tpu as pltpu
from jax.experimental.pallas import tpu_sc as plsc
from jax._src.pallas import mpmd as _mpmd

N_ROWS = 1000000
DIM = 64
N_UPD = 16384
NC = 2
NS = 16
NW = NC * NS
B_PER_W = N_UPD // NW
CHUNK = 128
N_CHUNK = B_PER_W // CHUNK
LANES = 16
M_COEF = 0.5
EPS = 1e-12


def _body(mem_alias, mem, x, yc, out, idx_v, rows_v, x_v, sem):
    del mem_alias
    wid = lax.axis_index("s") * NC + lax.axis_index("c")
    base = wid * B_PER_W

    pltpu.sync_copy(yc.at[wid], idx_v)
    pltpu.sync_copy(x.at[pl.ds(base, B_PER_W)], x_v)

    gathers = [
        pltpu.async_copy(
            mem.at[idx_v.at[j]], rows_v.at[pl.ds(j * CHUNK, CHUNK)], sem
        )
        for j in range(N_CHUNK)
    ]
    for cp in gathers:
        cp.wait()

    half = jnp.float32(M_COEF)

    def row_fn(r, _):
        w = [
            rows_v[r, pl.ds(j * LANES, LANES)] * half
            + x_v[r, pl.ds(j * LANES, LANES)] * half
            for j in range(DIM // LANES)
        ]
        ssq = w[0] * w[0]
        for wj in w[1:]:
            ssq = ssq + wj * wj
        lane = lax.iota(jnp.int32, LANES)
        for k in (8, 4, 2, 1):
            ssq = ssq + lax.gather(
                ssq,
                (lane ^ k)[:, None],
                lax.GatherDimensionNumbers(
                    offset_dims=(),
                    collapsed_slice_dims=(0,),
                    start_index_map=(0,),
                ),
                (1,),
                mode=lax.GatherScatterMode.PROMISE_IN_BOUNDS,
            )
        s = ssq
        bits = lax.bitcast_convert_type(s, jnp.int32)
        yv = lax.bitcast_convert_type(
            jnp.int32(0x5F3759DF) - (bits >> 1), jnp.float32
        )
        for _ in range(3):
            yv = yv * (jnp.float32(1.5) - jnp.float32(0.5) * s * yv * yv)
        norm = s * yv
        inv = jnp.float32(1.0) / jnp.maximum(norm, jnp.float32(EPS))
        for j in range(DIM // LANES):
            rows_v[r, pl.ds(j * LANES, LANES)] = w[j] * inv
        return 0

    lax.fori_loop(0, B_PER_W, row_fn, 0)

    scatters = [
        pltpu.async_copy(
            rows_v.at[pl.ds(j * CHUNK, CHUNK)], out.at[idx_v.at[j]], sem
        )
        for j in range(N_CHUNK)
    ]
    for cp in scatters:
        cp.wait()


def _make_update(interpret=False):
    mesh = plsc.VectorSubcoreMesh(core_axis_name="c", subcore_axis_name="s")
    return _mpmd._mpmd_map(
        [(mesh, _body)],
        out_types=jax.ShapeDtypeStruct((N_ROWS, DIM), jnp.float32),
        input_output_aliases={0: 0},
        scratch_types=[
            pltpu.VMEM((N_CHUNK, CHUNK), jnp.int32),
            pltpu.VMEM((B_PER_W, DIM), jnp.float32),
            pltpu.VMEM((B_PER_W, DIM), jnp.float32),
            pltpu.SemaphoreType.DMA,
        ],
        interpret=interpret,
        compiler_params=pltpu.CompilerParams(use_tc_tiling_on_sc=False),
        name="basemem_update_sc",
    )


_update = _make_update()


def kernel(memory, x, y):
    yc = y.astype(jnp.int32).reshape(NW, N_CHUNK, CHUNK)
    return _update(memory, memory, x, yc)


# speedup vs baseline: 2.5599x; 2.5599x over previous
import jax
import jax.numpy as jnp
from jax import lax
from jax.experimental import pallas as pl
from jax.experimental.pallas import tpu as pltpu
from jax.experimental.pallas import tpu_sc as plsc
from jax._src.pallas import mpmd as _mpmd

N_ROWS = 1000000
N_PAD = 1000064
DIM = 64
N_UPD = 16384
NC = 2
NS = 16
NW = NC * NS
LANES = 16
W = 512
CHUNKS_PER_TILE = 61
TILE_COLS = W * CHUNKS_PER_TILE
EXTRA0 = NW * TILE_COLS
TAILP0 = EXTRA0 + W
CAP = N_UPD + LANES
M_COEF = 0.5
EPS = 1e-12


def _prefix_and_total(m, lane):
    excl = jnp.zeros((LANES,), jnp.int32)
    for l in range(1, LANES):
        pc = plsc.all_reduce_population_count(m & (lane < l))
        excl = jnp.where(lane == l, pc, excl)
    return excl, plsc.all_reduce_population_count(m)[0]


def _newton_inv_norm(ssq):
    bits = lax.bitcast_convert_type(ssq, jnp.int32)
    yv = lax.bitcast_convert_type(
        jnp.int32(0x5F3759DF) - (bits >> 1), jnp.float32
    )
    for _ in range(3):
        yv = yv * (jnp.float32(1.5) - jnp.float32(0.5) * ssq * yv * yv)
    norm = ssq * yv
    return jnp.float32(1.0) / jnp.maximum(norm, jnp.float32(EPS))


def _body(mem, tailp, xflat, yh, out, ybuf, ibuf, kbuf, sibuf, skbuf, ch, xg,
          sem):
    wid = lax.axis_index("s") * NC + lax.axis_index("c")
    lo = wid * TILE_COLS
    hi = lo + TILE_COLS
    lane = lax.iota(jnp.int32, LANES)
    half = jnp.float32(M_COEF)
    is_tail_tile = wid == 0

    def scan_piece(p, cnt):
        pltpu.sync_copy(yh.at[pl.ds(p * 1024, 1024)], ybuf)

        def scan_vec(v, cnt):
            i16 = ybuf[pl.ds(v * LANES, LANES)]
            k16 = lane + (p * 1024 + v * LANES)
            m = (i16 >= lo) & (i16 < hi)
            m = m | ((i16 >= EXTRA0) & is_tail_tile)
            excl, tot = _prefix_and_total(m, lane)
            dest = jnp.where(m, cnt + excl, CAP - 1)
            plsc.store_scatter(ibuf, [dest], i16)
            plsc.store_scatter(kbuf, [dest], k16)
            return cnt + tot

        return lax.fori_loop(0, 1024 // LANES, scan_vec, cnt)

    cnt = lax.fori_loop(0, N_UPD // 1024, scan_piece, jnp.int32(0))
    n_grp = (cnt + LANES - 1) // LANES

    def apply_updates(c0, cw):
        def select(g, off):
            valid = (lane + g * LANES) < cnt
            i16 = ibuf[pl.ds(g * LANES, LANES)]
            k16 = kbuf[pl.ds(g * LANES, LANES)]
            m = valid & (i16 >= c0) & (i16 < c0 + cw)
            excl, tot = _prefix_and_total(m, lane)
            dest = jnp.where(m, off + excl, CAP - 1)
            plsc.store_scatter(sibuf, [dest], i16 - c0)
            plsc.store_scatter(skbuf, [dest], k16)
            return off + tot

        off = lax.fori_loop(0, n_grp, select, jnp.int32(0))

        def patch(g, _):
            in_m = (lane + g * LANES) < off
            cols = jnp.where(in_m, sibuf[pl.ds(g * LANES, LANES)], cw)
            ks = jnp.where(in_m, skbuf[pl.ds(g * LANES, LANES)], 0)
            fetches = []
            for l in range(LANES):
                kk = (ks[l] >> 1) << 1
                fetches.append(
                    pltpu.async_copy(
                        xflat.at[pl.ds(kk * DIM, 2 * DIM)], xg.at[l], sem
                    )
                )
            for cp in fetches:
                cp.wait()
            xcol0 = (ks & 1) * DIM
            ssq = jnp.zeros((LANES,), jnp.float32)
            for j in range(DIM):
                mv = plsc.load_gather(
                    ch, [jnp.full((LANES,), j, jnp.int32), cols]
                )
                xv = plsc.load_gather(xg, [lane, xcol0 + j])
                w = mv * half + xv * half
                plsc.store_scatter(xg, [lane, xcol0 + j], w)
                ssq = ssq + w * w
            inv = _newton_inv_norm(ssq)
            for j in range(DIM):
                w = plsc.load_gather(xg, [lane, xcol0 + j])
                plsc.store_scatter(
                    ch, [jnp.full((LANES,), j, jnp.int32), cols], w * inv
                )
            return 0

        lax.fori_loop(0, (off + LANES - 1) // LANES, patch, 0)

    def run_chunk(c0):
        pltpu.sync_copy(mem.at[:, pl.ds(c0, W)], ch.at[:, pl.ds(0, W)])
        apply_updates(c0, W)
        pltpu.sync_copy(ch.at[:, pl.ds(0, W)], out.at[:, pl.ds(c0, W)])

    n_chunks = jnp.where(is_tail_tile, CHUNKS_PER_TILE + 1, CHUNKS_PER_TILE)

    def chunk_loop(cidx, _):
        c0 = jnp.where(cidx == CHUNKS_PER_TILE, EXTRA0, lo + cidx * W)
        run_chunk(c0)
        return 0

    lax.fori_loop(0, n_chunks, chunk_loop, 0)

    pltpu.sync_copy(tailp, ch.at[:, pl.ds(0, 128)])
    apply_updates(jnp.int32(TAILP0), 128)

    @pl.when(is_tail_tile)
    def _():
        pltpu.sync_copy(ch.at[:, pl.ds(0, 128)], out.at[:, pl.ds(TAILP0, 128)])


def _make_update(interpret=False):
    mesh = plsc.VectorSubcoreMesh(core_axis_name="c", subcore_axis_name="s")
    return _mpmd.mpmd_map(
        [(mesh, _body)],
        out_types=jax.ShapeDtypeStruct((DIM, N_PAD), jnp.float32),
        scratch_types=[
            pltpu.VMEM((1024,), jnp.int32),
            pltpu.VMEM((CAP,), jnp.int32),
            pltpu.VMEM((CAP,), jnp.int32),
            pltpu.VMEM((CAP,), jnp.int32),
            pltpu.VMEM((CAP,), jnp.int32),
            pltpu.VMEM((DIM, W + 128), jnp.float32),
            pltpu.VMEM((LANES, 2 * DIM), jnp.float32),
            pltpu.SemaphoreType.DMA,
        ],
        interpret=interpret,
        compiler_params=pltpu.CompilerParams(
            use_tc_tiling_on_sc=True, needs_layout_passes=False
        ),
        name="basemem_update_sc",
    )


_update = _make_update()


def kernel(memory, x, y):
    mt = memory.T
    tailp = jnp.concatenate(
        [mt[:, TAILP0:], jnp.zeros((DIM, N_PAD - N_ROWS), jnp.float32)],
        axis=1,
    )
    xflat = x.reshape(-1)
    yc = y.astype(jnp.int32)
    out_pad = _update(mt, tailp, xflat, yc)
    return out_pad[:, :N_ROWS].T
